# baseline (speedup 1.0000x reference)
import numpy as np

import jax
import jax.numpy as jnp
from jax import lax
from jax.experimental import pallas as pl
from jax.experimental.pallas import tpu as pltpu
from jax.experimental.pallas import tpu_sc as plsc

B = 4
N = 16384
F = 128
MAX_GT = 8192
TOP_K = 3

L = 16
KV = F // L
CH = 128
RB = 16
NB = CH // RB
PW = 3 * L + 1
NW = 32
CHUNKS_PER_BATCH = MAX_GT // CH
TOTAL_CHUNKS = B * CHUNKS_PER_BATCH
STEPS = TOTAL_CHUNKS // NW

_NEG = np.float32(-3.4028234663852886e38)
_IMIN = np.int32(-2147483648)


def _sc_body(soft_hbm, params_hbm, out_hbm, pvec, vin, vout, negbuf, pool):
    wid = lax.axis_index("s") * 2 + lax.axis_index("c")
    pltpu.sync_copy(params_hbm, pvec)
    lanes = lax.iota(jnp.int32, L)
    pv = pvec[...]

    def extract(idx):
        return jnp.max(jnp.where(lanes == idx, pv, _IMIN))

    negv = jnp.full((L,), _NEG)
    none = jnp.full((L,), jnp.float32(-1.0))
    zero = jnp.zeros((L,), jnp.float32)

    def fill_neg(r, _):
        for k in range(KV):
            negbuf[r, pl.ds(L * k, L)] = none
        return 0

    lax.fori_loop(0, CH, fill_neg, 0)

    def insert3(a1, a2, a3, x):
        t1 = jnp.maximum(a1, x)
        c2 = jnp.minimum(a1, x)
        t2 = jnp.maximum(a2, c2)
        c3 = jnp.minimum(a2, c2)
        t3 = jnp.maximum(a3, c3)
        return t1, t2, t3

    def make_block_body(dshift):
        def block_body(bi, _):
            rbase = bi * RB
            rin = rbase + dshift
            for r in range(RB):
                x = [vin[rin + r, pl.ds(L * k, L)] for k in range(KV)]
                a1 = x[0]
                a2 = negv
                a3 = negv
                for k in range(1, KV):
                    a1, a2, a3 = insert3(a1, a2, a3, x[k])
                pool[r, pl.ds(0, L)] = a1
                pool[r, pl.ds(L, L)] = a2
                pool[r, pl.ds(2 * L, L)] = a3
            q1 = negv
            q2 = negv
            q3 = negv
            for j in range(3 * L):
                vj = plsc.load_gather(pool, [lanes, jnp.full((L,), j, jnp.int32)])
                q1, q2, q3 = insert3(q1, q2, q3, vj)
            dnums = lax.GatherDimensionNumbers(
                offset_dims=(), collapsed_slice_dims=(0,), start_index_map=(0,)
            )
            for r in range(RB):
                thr = lax.gather(
                    q3,
                    jnp.full((L, 1), r, jnp.int32),
                    dimension_numbers=dnums,
                    slice_sizes=(1,),
                    mode=lax.GatherScatterMode.PROMISE_IN_BOUNDS,
                )
                for k in range(KV):
                    xk = vin[rin + r, pl.ds(L * k, L)]
                    vout[rbase + r, pl.ds(L * k, L)] = jnp.where(
                        xk >= thr, xk, zero
                    )
            return 0

        return block_body

    def fill_row(r, _):
        for k in range(KV):
            vout[r, pl.ds(L * k, L)] = none
        return 0

    def chunk_body(t, _):
        even = (t % 2) == 0
        cg = jnp.where(even, t * NW + wid, t * NW + (NW - 1) - wid)
        bi = cg // CHUNKS_PER_BATCH
        cl = cg % CHUNKS_PER_BATCH
        start = extract(bi)
        cnt = extract(bi + B)
        src = start + cl * CH
        vc = jnp.clip(jnp.minimum(cnt, MAX_GT) - cl * CH, 0, CH)
        srcc = jnp.minimum(src, N - CH)
        dshift = src - srcc
        out_at = out_hbm.at[pl.ds(cg * CH, CH)]

        @pl.when(vc > 0)
        def _():
            pltpu.sync_copy(soft_hbm.at[pl.ds(srcc, CH)], vin)
            nblk = (vc + (RB - 1)) // RB
            lax.fori_loop(0, nblk, make_block_body(dshift), 0)
            lax.fori_loop(vc, CH, fill_row, 0)
            pltpu.sync_copy(vout, out_at)

        @pl.when(vc <= 0)
        def _():
            pltpu.sync_copy(negbuf, out_at)

        return 0

    lax.fori_loop(0, STEPS, chunk_body, 0)


@jax.jit
def kernel(soft_weight, gt_boxes_batch_ids, gt_boxes_count):
    del gt_boxes_batch_ids
    counts = gt_boxes_count[:, 0].astype(jnp.int32)
    starts = jnp.concatenate(
        [jnp.zeros((1,), jnp.int32), jnp.cumsum(counts)[:-1].astype(jnp.int32)]
    )
    params = jnp.concatenate(
        [starts, counts, jnp.zeros((L - 2 * B,), jnp.int32)]
    )

    mesh = plsc.VectorSubcoreMesh(core_axis_name="c", subcore_axis_name="s")
    out = pl.kernel(
        _sc_body,
        out_type=jax.ShapeDtypeStruct((B * MAX_GT, F), jnp.float32),
        mesh=mesh,
        compiler_params=pltpu.CompilerParams(
            use_tc_tiling_on_sc=False, needs_layout_passes=False
        ),
        scratch_types=[
            pltpu.VMEM((L,), jnp.int32),
            pltpu.VMEM((CH, F), jnp.float32),
            pltpu.VMEM((CH, F), jnp.float32),
            pltpu.VMEM((CH, F), jnp.float32),
            pltpu.VMEM((RB, PW), jnp.float32),
        ],
    )(soft_weight, params)
    return out.reshape(B, MAX_GT, F)


# speedup vs baseline: 1.2774x; 1.2774x over previous
import numpy as np

import jax
import jax.numpy as jnp
from jax import lax
from jax.experimental import pallas as pl
from jax.experimental.pallas import tpu as pltpu
from jax.experimental.pallas import tpu_sc as plsc

B = 4
N = 16384
F = 128
MAX_GT = 8192
TOP_K = 3

L = 16
KV = F // L
CH = 128
GR = 2
NW = 32
CHUNKS_PER_BATCH = MAX_GT // CH
TOTAL_CHUNKS = B * CHUNKS_PER_BATCH
STEPS = TOTAL_CHUNKS // NW

_NEG = np.float32(-3.4028234663852886e38)
_IMIN = np.int32(-2147483648)

_GATHER_DNUMS = lax.GatherDimensionNumbers(
    offset_dims=(), collapsed_slice_dims=(0,), start_index_map=(0,)
)


def _rot(x, idx):
    return lax.gather(
        x,
        idx,
        dimension_numbers=_GATHER_DNUMS,
        slice_sizes=(1,),
        mode=lax.GatherScatterMode.PROMISE_IN_BOUNDS,
    )


def _sc_body(soft_hbm, params_hbm, out_hbm, pvec, vin, vout, negbuf):
    wid = lax.axis_index("s") * 2 + lax.axis_index("c")
    pltpu.sync_copy(params_hbm, pvec)
    lanes = lax.iota(jnp.int32, L)
    pv = pvec[...]
    rotidx = [((lanes + s) & (L - 1)).reshape(L, 1) for s in (8, 4, 2, 1)]

    def extract(idx):
        return jnp.max(jnp.where(lanes == idx, pv, _IMIN))

    negv = jnp.full((L,), _NEG)
    none = jnp.full((L,), jnp.float32(-1.0))
    zero = jnp.zeros((L,), jnp.float32)

    def fill_neg(r, _):
        for k in range(KV):
            negbuf[r, pl.ds(L * k, L)] = none
        return 0

    lax.fori_loop(0, CH, fill_neg, 0)

    def row_compute(rin, rout):
        x = [vin[rin, pl.ds(L * k, L)] for k in range(KV)]
        a1 = x[0]
        a2 = negv
        a3 = negv
        for k in range(1, KV):
            t1 = jnp.maximum(a1, x[k])
            c2 = jnp.minimum(a1, x[k])
            t2 = jnp.maximum(a2, c2)
            c3 = jnp.minimum(a2, c2)
            a3 = jnp.maximum(a3, c3)
            a1 = t1
            a2 = t2
        for i, s in enumerate((8, 4, 2)):
            b1 = _rot(a1, rotidx[i])
            b2 = _rot(a2, rotidx[i])
            b3 = _rot(a3, rotidx[i])
            l1 = jnp.maximum(a1, b3)
            l2 = jnp.maximum(a2, b2)
            l3 = jnp.maximum(a3, b1)
            u = jnp.maximum(l1, l2)
            v = jnp.minimum(l1, l2)
            w = jnp.maximum(v, l3)
            xm = jnp.minimum(v, l3)
            a1 = jnp.maximum(u, w)
            a2 = jnp.minimum(u, w)
            a3 = xm
        b1 = _rot(a1, rotidx[3])
        b2 = _rot(a2, rotidx[3])
        b3 = _rot(a3, rotidx[3])
        l1 = jnp.maximum(a1, b3)
        l2 = jnp.maximum(a2, b2)
        l3 = jnp.maximum(a3, b1)
        thr = jnp.minimum(jnp.minimum(l1, l2), l3)
        for k in range(KV):
            vout[rout, pl.ds(L * k, L)] = jnp.where(x[k] >= thr, x[k], zero)

    def make_group_body(dshift):
        def group_body(g, _):
            r0 = g * GR
            for r in range(GR):
                row_compute(r0 + r + dshift, r0 + r)
            return 0

        return group_body

    def fill_row(r, _):
        for k in range(KV):
            vout[r, pl.ds(L * k, L)] = none
        return 0

    def chunk_body(t, _):
        even = (t % 2) == 0
        cg = jnp.where(even, t * NW + wid, t * NW + (NW - 1) - wid)
        bi = cg // CHUNKS_PER_BATCH
        cl = cg % CHUNKS_PER_BATCH
        start = extract(bi)
        cnt = extract(bi + B)
        src = start + cl * CH
        vc = jnp.clip(jnp.minimum(cnt, MAX_GT) - cl * CH, 0, CH)
        srcc = jnp.minimum(src, N - CH)
        dshift = src - srcc
        out_at = out_hbm.at[pl.ds(cg * CH, CH)]

        @pl.when(vc > 0)
        def _():
            pltpu.sync_copy(soft_hbm.at[pl.ds(srcc, CH)], vin.at[pl.ds(0, CH)])
            ngrp = (vc + (GR - 1)) // GR
            lax.fori_loop(0, ngrp, make_group_body(dshift), 0)
            lax.fori_loop(vc, CH, fill_row, 0)
            pltpu.sync_copy(vout, out_at)

        @pl.when(vc <= 0)
        def _():
            pltpu.sync_copy(negbuf, out_at)

        return 0

    lax.fori_loop(0, STEPS, chunk_body, 0)


@jax.jit
def kernel(soft_weight, gt_boxes_batch_ids, gt_boxes_count):
    del gt_boxes_batch_ids
    counts = gt_boxes_count[:, 0].astype(jnp.int32)
    starts = jnp.concatenate(
        [jnp.zeros((1,), jnp.int32), jnp.cumsum(counts)[:-1].astype(jnp.int32)]
    )
    params = jnp.concatenate(
        [starts, counts, jnp.zeros((L - 2 * B,), jnp.int32)]
    )

    mesh = plsc.VectorSubcoreMesh(core_axis_name="c", subcore_axis_name="s")
    out = pl.kernel(
        _sc_body,
        out_type=jax.ShapeDtypeStruct((B * MAX_GT, F), jnp.float32),
        mesh=mesh,
        compiler_params=pltpu.CompilerParams(
            use_tc_tiling_on_sc=False, needs_layout_passes=False
        ),
        scratch_types=[
            pltpu.VMEM((L,), jnp.int32),
            pltpu.VMEM((CH + 1, F), jnp.float32),
            pltpu.VMEM((CH, F), jnp.float32),
            pltpu.VMEM((CH, F), jnp.float32),
        ],
    )(soft_weight, params)
    return out.reshape(B, MAX_GT, F)
